# baseline (speedup 1.0000x reference)
import functools

import jax
import jax.numpy as jnp
from jax import lax
from jax.experimental import pallas as pl
from jax.experimental.pallas import tpu as pltpu
from jax.experimental.pallas import tpu_sc as plsc

N = 10000
D = 128
E = 320000
C = 64

NC = 2
NS = 16
NW = NC * NS
PT = E // NW
CH = 40
NCH = PT // CH
CW = 16
RT0 = 624
RTW = 640
ZR = 32


def _seg_sum_sc(width, with_cnt):
  mesh = plsc.VectorSubcoreMesh(core_axis_name="c", subcore_axis_name="s")
  out_type = [jax.ShapeDtypeStruct((NC, N, width), jnp.float32)]
  scratch = [
      pltpu.VMEM((NCH, CH), jnp.int32),
      pltpu.VMEM((NCH, CH), jnp.int32),
      pltpu.VMEM((CH, width), jnp.float32),
      pltpu.VMEM((CH, width), jnp.float32),
      pltpu.VMEM((ZR, width), jnp.float32),
      pltpu.VMEM_SHARED((N, width), jnp.float32),
      pltpu.SemaphoreType.DMA,
      pltpu.SemaphoreType.DMA,
      pltpu.SemaphoreType.DMA,
      pltpu.SemaphoreType.DMA,
  ]
  if with_cnt:
    out_type.append(jax.ShapeDtypeStruct((NC, N, CW), jnp.float32))
    scratch += [
        pltpu.VMEM((CH, CW), jnp.float32),
        pltpu.VMEM((ZR, CW), jnp.float32),
        pltpu.VMEM_SHARED((N, CW), jnp.float32),
    ]

  def body(table_hbm, src_hbm, dst_hbm, *refs):
    if with_cnt:
      (agg_out, cnt_out, src_v, dst_v, rows_a, rows_b, zb_v, agg_sh, gs_a,
       gs_b, ss_a, ss_b, ones_v, zc_v, cnt_sh) = refs
    else:
      (agg_out, src_v, dst_v, rows_a, rows_b, zb_v, agg_sh, gs_a, gs_b,
       ss_a, ss_b) = refs
    cid = lax.axis_index("c")
    sid = lax.axis_index("s")
    wid = cid * NS + sid

    z16 = jnp.zeros((16,), jnp.float32)

    @pl.loop(0, ZR)
    def _(i):
      for j in range(width // 16):
        zb_v[i, pl.ds(j * 16, 16)] = z16

    if with_cnt:
      o16 = jnp.ones((16,), jnp.float32)

      @pl.loop(0, ZR)
      def _(i):
        zc_v[i, :] = z16

      @pl.loop(0, CH)
      def _(i):
        ones_v[i, :] = o16

    for k in range(RTW // ZR):
      pltpu.sync_copy(zb_v, agg_sh.at[pl.ds(sid * RT0 + k * ZR, ZR)])
      if with_cnt:
        pltpu.sync_copy(zc_v, cnt_sh.at[pl.ds(sid * RT0 + k * ZR, ZR)])

    pltpu.sync_copy(src_hbm.at[wid], src_v)
    pltpu.sync_copy(dst_hbm.at[wid], dst_v)

    plsc.subcore_barrier()

    def start_gather(rows, gs, j):
      pltpu.async_copy(table_hbm.at[src_v.at[j]], rows, gs)

    def wait_gather(rows, gs, j):
      pltpu.make_async_copy(table_hbm.at[src_v.at[j]], rows, gs).wait()

    def start_scatter(rows, ss, j):
      pltpu.async_copy(rows, agg_sh.at[dst_v.at[j]], ss, add=True)
      if with_cnt:
        pltpu.async_copy(ones_v, cnt_sh.at[dst_v.at[j]], ss, add=True)

    def wait_scatter(rows, ss, j):
      pltpu.make_async_copy(rows, agg_sh.at[dst_v.at[j]], ss).wait()
      if with_cnt:
        pltpu.make_async_copy(ones_v, cnt_sh.at[dst_v.at[j]], ss).wait()

    start_gather(rows_a, gs_a, 0)
    start_gather(rows_b, gs_b, 1)
    wait_gather(rows_a, gs_a, 0)
    start_scatter(rows_a, ss_a, 0)

    @pl.loop(0, NCH - 3, step=2)
    def _(j):
      wait_gather(rows_b, gs_b, j + 1)
      start_scatter(rows_b, ss_b, j + 1)
      wait_scatter(rows_a, ss_a, j)
      start_gather(rows_a, gs_a, j + 2)
      wait_gather(rows_a, gs_a, j + 2)
      start_scatter(rows_a, ss_a, j + 2)
      wait_scatter(rows_b, ss_b, j + 1)
      start_gather(rows_b, gs_b, j + 3)

    wait_gather(rows_b, gs_b, NCH - 1)
    start_scatter(rows_b, ss_b, NCH - 1)
    wait_scatter(rows_a, ss_a, NCH - 2)
    wait_scatter(rows_b, ss_b, NCH - 1)

    plsc.subcore_barrier()

    pltpu.sync_copy(agg_sh.at[pl.ds(sid * RT0, RTW)],
                    agg_out.at[cid, pl.ds(sid * RT0, RTW)])
    if with_cnt:
      pltpu.sync_copy(cnt_sh.at[pl.ds(sid * RT0, RTW)],
                      cnt_out.at[cid, pl.ds(sid * RT0, RTW)])

  return pl.kernel(
      body, out_type=out_type, mesh=mesh, scratch_types=scratch,
      compiler_params=pltpu.CompilerParams(use_tc_tiling_on_sc=False))


_seg_sum_128 = _seg_sum_sc(D, with_cnt=True)
_seg_sum_64 = _seg_sum_sc(C, with_cnt=False)

_TC_R = 1000


def _layer0_body(agg_ref, cnt_ref, x_ref, wl0_ref, bl0_ref, wr0_ref, wl1_ref,
                 h_ref, y1_ref):
  agg = agg_ref[0] + agg_ref[1]
  cnt = cnt_ref[0, :, 0:1] + cnt_ref[1, :, 0:1]
  mean = agg / jnp.maximum(cnt, 1.0)
  pre = (jnp.dot(mean, wl0_ref[...], preferred_element_type=jnp.float32)
         + bl0_ref[...]
         + jnp.dot(x_ref[...], wr0_ref[...], preferred_element_type=jnp.float32))
  h = jnp.maximum(pre, 0.0)
  h_ref[...] = h
  y1_ref[...] = jnp.dot(h, wl1_ref[...], preferred_element_type=jnp.float32)


def _layer1_body(agg_ref, cnt_ref, h_ref, wr1_ref, bl1_ref, out_ref):
  agg = agg_ref[0] + agg_ref[1]
  cnt = cnt_ref[0, :, 0:1] + cnt_ref[1, :, 0:1]
  pre = (agg / jnp.maximum(cnt, 1.0) + bl1_ref[...]
         + jnp.dot(h_ref[...], wr1_ref[...], preferred_element_type=jnp.float32))
  a = jnp.maximum(pre, 0.0)
  m = jnp.max(a, axis=-1, keepdims=True)
  lse = jnp.log(jnp.sum(jnp.exp(a - m), axis=-1, keepdims=True)) + m
  out_ref[...] = a - lse


_layer0 = pl.pallas_call(
    _layer0_body,
    grid=(N // _TC_R,),
    in_specs=[
        pl.BlockSpec((NC, _TC_R, D), lambda i: (0, i, 0)),
        pl.BlockSpec((NC, _TC_R, CW), lambda i: (0, i, 0)),
        pl.BlockSpec((_TC_R, D), lambda i: (i, 0)),
        pl.BlockSpec((D, D), lambda i: (0, 0)),
        pl.BlockSpec((1, D), lambda i: (0, 0)),
        pl.BlockSpec((D, D), lambda i: (0, 0)),
        pl.BlockSpec((D, C), lambda i: (0, 0)),
    ],
    out_specs=[
        pl.BlockSpec((_TC_R, D), lambda i: (i, 0)),
        pl.BlockSpec((_TC_R, C), lambda i: (i, 0)),
    ],
    out_shape=[
        jax.ShapeDtypeStruct((N, D), jnp.float32),
        jax.ShapeDtypeStruct((N, C), jnp.float32),
    ],
)

_layer1 = pl.pallas_call(
    _layer1_body,
    grid=(N // _TC_R,),
    in_specs=[
        pl.BlockSpec((NC, _TC_R, C), lambda i: (0, i, 0)),
        pl.BlockSpec((NC, _TC_R, CW), lambda i: (0, i, 0)),
        pl.BlockSpec((_TC_R, D), lambda i: (i, 0)),
        pl.BlockSpec((D, C), lambda i: (0, 0)),
        pl.BlockSpec((1, C), lambda i: (0, 0)),
    ],
    out_specs=pl.BlockSpec((_TC_R, C), lambda i: (i, 0)),
    out_shape=jax.ShapeDtypeStruct((N, C), jnp.float32),
)


@jax.jit
def kernel(x, edge_index, W_l0, b_l0, W_r0, W_l1, b_l1, W_r1):
  src = edge_index[0].reshape(NW, NCH, CH)
  dst = edge_index[1].reshape(NW, NCH, CH)

  agg0, cnt = _seg_sum_128(x, src, dst)
  h, y1 = _layer0(agg0, cnt, x, W_l0, b_l0.reshape(1, D), W_r0, W_l1)
  (agg1,) = _seg_sum_64(y1, src, dst)
  return _layer1(agg1, cnt, h, W_r1, b_l1.reshape(1, C))


# speedup vs baseline: 1.7113x; 1.7113x over previous
import functools

import jax
import jax.numpy as jnp
from jax import lax
from jax.experimental import pallas as pl
from jax.experimental.pallas import tpu as pltpu
from jax.experimental.pallas import tpu_sc as plsc

N = 10000
D = 128
E = 320000
C = 64

NC = 2
NS = 16
NW = NC * NS
PT = E // NW
CH = 80
NCH = PT // CH
CW = 8
RT0 = 624
RTW = 640
ZR = 32


def _seg_sum_sc(width, with_cnt):
  mesh = plsc.VectorSubcoreMesh(core_axis_name="c", subcore_axis_name="s")
  out_type = [jax.ShapeDtypeStruct((NC, N, width), jnp.float32)]
  scratch = [
      pltpu.VMEM((NCH, CH), jnp.int32),
      pltpu.VMEM((NCH, CH), jnp.int32),
      pltpu.VMEM((CH, width), jnp.float32),
      pltpu.VMEM((CH, width), jnp.float32),
      pltpu.VMEM((ZR, width), jnp.float32),
      pltpu.VMEM_SHARED((N, width), jnp.float32),
      pltpu.SemaphoreType.DMA,
      pltpu.SemaphoreType.DMA,
  ]
  if with_cnt:
    out_type.append(jax.ShapeDtypeStruct((NC, N, CW), jnp.float32))
    scratch += [
        pltpu.VMEM((CH, CW), jnp.float32),
        pltpu.VMEM_SHARED((N, CW), jnp.float32),
    ]

  def body(table_hbm, src_hbm, dst_hbm, ones_hbm, zeros_hbm, *refs):
    if with_cnt:
      (agg_out, cnt_out, src_v, dst_v, rows_a, rows_b, zb_v, agg_sh, sem_a,
       sem_b, ones_v, cnt_sh) = refs
    else:
      (agg_out, src_v, dst_v, rows_a, rows_b, zb_v, agg_sh, sem_a,
       sem_b) = refs
    cid = lax.axis_index("c")
    sid = lax.axis_index("s")
    wid = cid * NS + sid

    z16 = jnp.zeros((16,), jnp.float32)

    @pl.loop(0, ZR)
    def _(i):
      for j in range(width // 16):
        zb_v[i, pl.ds(j * 16, 16)] = z16

    for k in range(RTW // ZR):
      pltpu.sync_copy(zb_v, agg_sh.at[pl.ds(sid * RT0 + k * ZR, ZR)])
    if with_cnt:
      pltpu.sync_copy(ones_hbm, ones_v)
      pltpu.sync_copy(zeros_hbm.at[pl.ds(sid * RT0, RTW)],
                      cnt_sh.at[pl.ds(sid * RT0, RTW)])

    pltpu.sync_copy(src_hbm.at[wid], src_v)
    pltpu.sync_copy(dst_hbm.at[wid], dst_v)

    plsc.subcore_barrier()

    def start_gather(rows, sem, j):
      pltpu.async_copy(table_hbm.at[src_v.at[j]], rows, sem)

    def wait_gather(rows, sem, j):
      pltpu.make_async_copy(table_hbm.at[src_v.at[j]], rows, sem).wait()

    def scatter(rows, j):
      pltpu.sync_copy(rows, agg_sh.at[dst_v.at[j]], add=True)
      if with_cnt:
        pltpu.sync_copy(ones_v, cnt_sh.at[dst_v.at[j]], add=True)

    start_gather(rows_a, sem_a, 0)

    @pl.loop(1, NCH - 1, step=2)
    def _(j):
      start_gather(rows_b, sem_b, j)
      wait_gather(rows_a, sem_a, j - 1)
      scatter(rows_a, j - 1)
      start_gather(rows_a, sem_a, j + 1)
      wait_gather(rows_b, sem_b, j)
      scatter(rows_b, j)

    wait_gather(rows_a, sem_a, NCH - 1)
    scatter(rows_a, NCH - 1)

    plsc.subcore_barrier()

    pltpu.sync_copy(agg_sh.at[pl.ds(sid * RT0, RTW)],
                    agg_out.at[cid, pl.ds(sid * RT0, RTW)])
    if with_cnt:
      pltpu.sync_copy(cnt_sh.at[pl.ds(sid * RT0, RTW)],
                      cnt_out.at[cid, pl.ds(sid * RT0, RTW)])

  return pl.kernel(
      body, out_type=out_type, mesh=mesh, scratch_types=scratch,
      compiler_params=pltpu.CompilerParams(use_tc_tiling_on_sc=False))


_seg_sum_128 = _seg_sum_sc(D, with_cnt=True)
_seg_sum_64 = _seg_sum_sc(C, with_cnt=False)

_TC_R = 1000


def _layer0_body(agg_ref, cnt_ref, x_ref, wl0_ref, bl0_ref, wr0_ref, wl1_ref,
                 h_ref, y1_ref):
  agg = agg_ref[0] + agg_ref[1]
  cnt = cnt_ref[0, :, 0:1] + cnt_ref[1, :, 0:1]
  mean = agg / jnp.maximum(cnt, 1.0)
  pre = (jnp.dot(mean, wl0_ref[...], preferred_element_type=jnp.float32)
         + bl0_ref[...]
         + jnp.dot(x_ref[...], wr0_ref[...], preferred_element_type=jnp.float32))
  h = jnp.maximum(pre, 0.0)
  h_ref[...] = h
  y1_ref[...] = jnp.dot(h, wl1_ref[...], preferred_element_type=jnp.float32)


def _layer1_body(agg_ref, cnt_ref, h_ref, wr1_ref, bl1_ref, out_ref):
  agg = agg_ref[0] + agg_ref[1]
  cnt = cnt_ref[0, :, 0:1] + cnt_ref[1, :, 0:1]
  pre = (agg / jnp.maximum(cnt, 1.0) + bl1_ref[...]
         + jnp.dot(h_ref[...], wr1_ref[...], preferred_element_type=jnp.float32))
  a = jnp.maximum(pre, 0.0)
  m = jnp.max(a, axis=-1, keepdims=True)
  lse = jnp.log(jnp.sum(jnp.exp(a - m), axis=-1, keepdims=True)) + m
  out_ref[...] = a - lse


_layer0 = pl.pallas_call(
    _layer0_body,
    grid=(N // _TC_R,),
    in_specs=[
        pl.BlockSpec((NC, _TC_R, D), lambda i: (0, i, 0)),
        pl.BlockSpec((NC, _TC_R, CW), lambda i: (0, i, 0)),
        pl.BlockSpec((_TC_R, D), lambda i: (i, 0)),
        pl.BlockSpec((D, D), lambda i: (0, 0)),
        pl.BlockSpec((1, D), lambda i: (0, 0)),
        pl.BlockSpec((D, D), lambda i: (0, 0)),
        pl.BlockSpec((D, C), lambda i: (0, 0)),
    ],
    out_specs=[
        pl.BlockSpec((_TC_R, D), lambda i: (i, 0)),
        pl.BlockSpec((_TC_R, C), lambda i: (i, 0)),
    ],
    out_shape=[
        jax.ShapeDtypeStruct((N, D), jnp.float32),
        jax.ShapeDtypeStruct((N, C), jnp.float32),
    ],
)

_layer1 = pl.pallas_call(
    _layer1_body,
    grid=(N // _TC_R,),
    in_specs=[
        pl.BlockSpec((NC, _TC_R, C), lambda i: (0, i, 0)),
        pl.BlockSpec((NC, _TC_R, CW), lambda i: (0, i, 0)),
        pl.BlockSpec((_TC_R, D), lambda i: (i, 0)),
        pl.BlockSpec((D, C), lambda i: (0, 0)),
        pl.BlockSpec((1, C), lambda i: (0, 0)),
    ],
    out_specs=pl.BlockSpec((_TC_R, C), lambda i: (i, 0)),
    out_shape=jax.ShapeDtypeStruct((N, C), jnp.float32),
)


@jax.jit
def kernel(x, edge_index, W_l0, b_l0, W_r0, W_l1, b_l1, W_r1):
  src = edge_index[0].reshape(NW, NCH, CH)
  dst = edge_index[1].reshape(NW, NCH, CH)
  ones = jnp.ones((CH, CW), jnp.float32)
  zeros = jnp.zeros((N, CW), jnp.float32)

  agg0, cnt = _seg_sum_128(x, src, dst, ones, zeros)
  h, y1 = _layer0(agg0, cnt, x, W_l0, b_l0.reshape(1, D), W_r0, W_l1)
  (agg1,) = _seg_sum_64(y1, src, dst, ones, zeros)
  return _layer1(agg1, cnt, h, W_r1, b_l1.reshape(1, C))


# speedup vs baseline: 1.7878x; 1.0447x over previous
import functools

import jax
import jax.numpy as jnp
from jax import lax
from jax.experimental import pallas as pl
from jax.experimental.pallas import tpu as pltpu
from jax.experimental.pallas import tpu_sc as plsc

N = 10000
D = 128
E = 320000
C = 64

NC = 2
NS = 16
NW = NC * NS
PT = E // NW
CH = 80
NCH = PT // CH
CH1 = 125
NCH1 = PT // CH1
CW = 8
RT0 = 624
RTW = 640
ZR = 32


def _seg_sum_sc(width, with_cnt, ch, nch):
  mesh = plsc.VectorSubcoreMesh(core_axis_name="c", subcore_axis_name="s")
  out_type = [jax.ShapeDtypeStruct((NC, N, width), jnp.float32)]
  scratch = [
      pltpu.VMEM((nch, ch), jnp.int32),
      pltpu.VMEM((nch, ch), jnp.int32),
      pltpu.VMEM((ch, width), jnp.float32),
      pltpu.VMEM((ch, width), jnp.float32),
      pltpu.VMEM((ZR, width), jnp.float32),
      pltpu.VMEM_SHARED((N, width), jnp.float32),
      pltpu.SemaphoreType.DMA,
      pltpu.SemaphoreType.DMA,
  ]
  if with_cnt:
    out_type.append(jax.ShapeDtypeStruct((NC, N, CW), jnp.float32))
    scratch += [
        pltpu.VMEM((ch, CW), jnp.float32),
        pltpu.VMEM_SHARED((N, CW), jnp.float32),
    ]

  def body(table_hbm, src_hbm, dst_hbm, ones_hbm, zeros_hbm, *refs):
    if with_cnt:
      (agg_out, cnt_out, src_v, dst_v, rows_a, rows_b, zb_v, agg_sh, sem_a,
       sem_b, ones_v, cnt_sh) = refs
    else:
      (agg_out, src_v, dst_v, rows_a, rows_b, zb_v, agg_sh, sem_a,
       sem_b) = refs
    cid = lax.axis_index("c")
    sid = lax.axis_index("s")
    wid = cid * NS + sid

    z16 = jnp.zeros((16,), jnp.float32)

    @pl.loop(0, ZR)
    def _(i):
      for j in range(width // 16):
        zb_v[i, pl.ds(j * 16, 16)] = z16

    for k in range(RTW // ZR):
      pltpu.sync_copy(zb_v, agg_sh.at[pl.ds(sid * RT0 + k * ZR, ZR)])
    if with_cnt:
      pltpu.sync_copy(ones_hbm, ones_v)
      pltpu.sync_copy(zeros_hbm.at[pl.ds(sid * RT0, RTW)],
                      cnt_sh.at[pl.ds(sid * RT0, RTW)])

    pltpu.sync_copy(src_hbm.at[wid], src_v)
    pltpu.sync_copy(dst_hbm.at[wid], dst_v)

    plsc.subcore_barrier()

    def start_gather(rows, sem, j):
      pltpu.async_copy(table_hbm.at[src_v.at[j]], rows, sem)

    def wait_gather(rows, sem, j):
      pltpu.make_async_copy(table_hbm.at[src_v.at[j]], rows, sem).wait()

    def scatter(rows, j):
      pltpu.sync_copy(rows, agg_sh.at[dst_v.at[j]], add=True)
      if with_cnt:
        pltpu.sync_copy(ones_v, cnt_sh.at[dst_v.at[j]], add=True)

    if nch % 2:
      start_gather(rows_a, sem_a, 0)

      @pl.loop(1, nch - 1, step=2)
      def _(j):
        start_gather(rows_b, sem_b, j)
        wait_gather(rows_a, sem_a, j - 1)
        scatter(rows_a, j - 1)
        start_gather(rows_a, sem_a, j + 1)
        wait_gather(rows_b, sem_b, j)
        scatter(rows_b, j)

      wait_gather(rows_a, sem_a, nch - 1)
      scatter(rows_a, nch - 1)
    else:
      start_gather(rows_a, sem_a, 0)
      start_gather(rows_b, sem_b, 1)

      @pl.loop(0, nch - 3, step=2)
      def _(j):
        wait_gather(rows_a, sem_a, j)
        scatter(rows_a, j)
        start_gather(rows_a, sem_a, j + 2)
        wait_gather(rows_b, sem_b, j + 1)
        scatter(rows_b, j + 1)
        start_gather(rows_b, sem_b, j + 3)

      wait_gather(rows_a, sem_a, nch - 2)
      scatter(rows_a, nch - 2)
      wait_gather(rows_b, sem_b, nch - 1)
      scatter(rows_b, nch - 1)

    plsc.subcore_barrier()

    pltpu.sync_copy(agg_sh.at[pl.ds(sid * RT0, RTW)],
                    agg_out.at[cid, pl.ds(sid * RT0, RTW)])
    if with_cnt:
      pltpu.sync_copy(cnt_sh.at[pl.ds(sid * RT0, RTW)],
                      cnt_out.at[cid, pl.ds(sid * RT0, RTW)])

  return pl.kernel(
      body, out_type=out_type, mesh=mesh, scratch_types=scratch,
      compiler_params=pltpu.CompilerParams(use_tc_tiling_on_sc=False))


_seg_sum_128 = _seg_sum_sc(D, with_cnt=True, ch=CH, nch=NCH)
_seg_sum_64 = _seg_sum_sc(C, with_cnt=False, ch=CH1, nch=NCH1)

_TC_R = 1000


def _layer0_body(agg_ref, cnt_ref, x_ref, wl0_ref, bl0_ref, wr0_ref, wl1_ref,
                 h_ref, y1_ref):
  agg = agg_ref[0] + agg_ref[1]
  cnt = cnt_ref[0, :, 0:1] + cnt_ref[1, :, 0:1]
  mean = agg / jnp.maximum(cnt, 1.0)
  pre = (jnp.dot(mean, wl0_ref[...], preferred_element_type=jnp.float32)
         + bl0_ref[...]
         + jnp.dot(x_ref[...], wr0_ref[...], preferred_element_type=jnp.float32))
  h = jnp.maximum(pre, 0.0)
  h_ref[...] = h
  y1_ref[...] = jnp.dot(h, wl1_ref[...], preferred_element_type=jnp.float32)


def _layer1_body(agg_ref, cnt_ref, h_ref, wr1_ref, bl1_ref, out_ref):
  agg = agg_ref[0] + agg_ref[1]
  cnt = cnt_ref[0, :, 0:1] + cnt_ref[1, :, 0:1]
  pre = (agg / jnp.maximum(cnt, 1.0) + bl1_ref[...]
         + jnp.dot(h_ref[...], wr1_ref[...], preferred_element_type=jnp.float32))
  a = jnp.maximum(pre, 0.0)
  m = jnp.max(a, axis=-1, keepdims=True)
  lse = jnp.log(jnp.sum(jnp.exp(a - m), axis=-1, keepdims=True)) + m
  out_ref[...] = a - lse


_layer0 = pl.pallas_call(
    _layer0_body,
    grid=(N // _TC_R,),
    in_specs=[
        pl.BlockSpec((NC, _TC_R, D), lambda i: (0, i, 0)),
        pl.BlockSpec((NC, _TC_R, CW), lambda i: (0, i, 0)),
        pl.BlockSpec((_TC_R, D), lambda i: (i, 0)),
        pl.BlockSpec((D, D), lambda i: (0, 0)),
        pl.BlockSpec((1, D), lambda i: (0, 0)),
        pl.BlockSpec((D, D), lambda i: (0, 0)),
        pl.BlockSpec((D, C), lambda i: (0, 0)),
    ],
    out_specs=[
        pl.BlockSpec((_TC_R, D), lambda i: (i, 0)),
        pl.BlockSpec((_TC_R, C), lambda i: (i, 0)),
    ],
    out_shape=[
        jax.ShapeDtypeStruct((N, D), jnp.float32),
        jax.ShapeDtypeStruct((N, C), jnp.float32),
    ],
)

_layer1 = pl.pallas_call(
    _layer1_body,
    grid=(N // _TC_R,),
    in_specs=[
        pl.BlockSpec((NC, _TC_R, C), lambda i: (0, i, 0)),
        pl.BlockSpec((NC, _TC_R, CW), lambda i: (0, i, 0)),
        pl.BlockSpec((_TC_R, D), lambda i: (i, 0)),
        pl.BlockSpec((D, C), lambda i: (0, 0)),
        pl.BlockSpec((1, C), lambda i: (0, 0)),
    ],
    out_specs=pl.BlockSpec((_TC_R, C), lambda i: (i, 0)),
    out_shape=jax.ShapeDtypeStruct((N, C), jnp.float32),
)


@jax.jit
def kernel(x, edge_index, W_l0, b_l0, W_r0, W_l1, b_l1, W_r1):
  src = edge_index[0].reshape(NW, NCH, CH)
  dst = edge_index[1].reshape(NW, NCH, CH)
  src1 = edge_index[0].reshape(NW, NCH1, CH1)
  dst1 = edge_index[1].reshape(NW, NCH1, CH1)
  ones = jnp.ones((CH, CW), jnp.float32)
  zeros = jnp.zeros((N, CW), jnp.float32)

  agg0, cnt = _seg_sum_128(x, src, dst, ones, zeros)
  h, y1 = _layer0(agg0, cnt, x, W_l0, b_l0.reshape(1, D), W_r0, W_l1)
  (agg1,) = _seg_sum_64(y1, src1, dst1, ones, zeros)
  return _layer1(agg1, cnt, h, W_r1, b_l1.reshape(1, C))


# speedup vs baseline: 1.8966x; 1.0609x over previous
import functools

import jax
import jax.numpy as jnp
from jax import lax
from jax.experimental import pallas as pl
from jax.experimental.pallas import tpu as pltpu
from jax.experimental.pallas import tpu_sc as plsc

N = 10000
D = 128
E = 320000
C = 64

NC = 2
NS = 16
NW = NC * NS
PT = E // NW
CH = 80
NCH = PT // CH
CH1 = 125
NCH1 = PT // CH1
CW = 8
RT0 = 624
RTW = 640
ZR = 32


def _seg_sum_sc(width, with_cnt, ch, nch, nbuf=2):
  odd2 = nbuf == 2 and nch % 2 == 1
  assert odd2 or nch % nbuf == 0
  mesh = plsc.VectorSubcoreMesh(core_axis_name="c", subcore_axis_name="s")
  out_type = [jax.ShapeDtypeStruct((NC, N, width), jnp.float32)]
  scratch = [
      pltpu.VMEM((nch, ch), jnp.int32),
      pltpu.VMEM((nch, ch), jnp.int32),
  ]
  scratch += [pltpu.VMEM((ch, width), jnp.float32)] * nbuf
  scratch += [
      pltpu.VMEM((ZR, width), jnp.float32),
      pltpu.VMEM_SHARED((N, width), jnp.float32),
  ]
  scratch += [pltpu.SemaphoreType.DMA] * nbuf
  if with_cnt:
    out_type.append(jax.ShapeDtypeStruct((NC, N, CW), jnp.float32))
    scratch += [
        pltpu.VMEM((ch, CW), jnp.float32),
        pltpu.VMEM_SHARED((N, CW), jnp.float32),
    ]

  def body(table_hbm, src_hbm, dst_hbm, ones_hbm, zeros_hbm, *refs):
    if with_cnt:
      agg_out, cnt_out = refs[0], refs[1]
      refs = refs[2:]
    else:
      agg_out = refs[0]
      refs = refs[1:]
    src_v, dst_v = refs[0], refs[1]
    rows = list(refs[2:2 + nbuf])
    zb_v, agg_sh = refs[2 + nbuf], refs[3 + nbuf]
    sems = list(refs[4 + nbuf:4 + 2 * nbuf])
    if with_cnt:
      ones_v, cnt_sh = refs[4 + 2 * nbuf], refs[5 + 2 * nbuf]
    cid = lax.axis_index("c")
    sid = lax.axis_index("s")
    wid = cid * NS + sid

    z16 = jnp.zeros((16,), jnp.float32)

    @pl.loop(0, ZR)
    def _(i):
      for j in range(width // 16):
        zb_v[i, pl.ds(j * 16, 16)] = z16

    for k in range(RTW // ZR):
      pltpu.sync_copy(zb_v, agg_sh.at[pl.ds(sid * RT0 + k * ZR, ZR)])
    if with_cnt:
      pltpu.sync_copy(ones_hbm, ones_v)
      pltpu.sync_copy(zeros_hbm.at[pl.ds(sid * RT0, RTW)],
                      cnt_sh.at[pl.ds(sid * RT0, RTW)])

    pltpu.sync_copy(src_hbm.at[wid], src_v)
    pltpu.sync_copy(dst_hbm.at[wid], dst_v)

    plsc.subcore_barrier()

    def start_gather(rows, sem, j):
      pltpu.async_copy(table_hbm.at[src_v.at[j]], rows, sem)

    def wait_gather(rows, sem, j):
      pltpu.make_async_copy(table_hbm.at[src_v.at[j]], rows, sem).wait()

    def scatter(rows, j):
      pltpu.sync_copy(rows, agg_sh.at[dst_v.at[j]], add=True)
      if with_cnt:
        pltpu.sync_copy(ones_v, cnt_sh.at[dst_v.at[j]], add=True)

    if odd2:
      start_gather(rows[0], sems[0], 0)

      @pl.loop(1, nch - 1, step=2)
      def _(j):
        start_gather(rows[1], sems[1], j)
        wait_gather(rows[0], sems[0], j - 1)
        scatter(rows[0], j - 1)
        start_gather(rows[0], sems[0], j + 1)
        wait_gather(rows[1], sems[1], j)
        scatter(rows[1], j)

      wait_gather(rows[0], sems[0], nch - 1)
      scatter(rows[0], nch - 1)
    else:
      for k in range(nbuf):
        start_gather(rows[k], sems[k], k)

      @pl.loop(0, nch - 2 * nbuf + 1, step=nbuf)
      def _(j):
        for k in range(nbuf):
          wait_gather(rows[k], sems[k], j + k)
          scatter(rows[k], j + k)
          start_gather(rows[k], sems[k], j + k + nbuf)

      for k in range(nbuf):
        wait_gather(rows[k], sems[k], nch - nbuf + k)
        scatter(rows[k], nch - nbuf + k)

    plsc.subcore_barrier()

    pltpu.sync_copy(agg_sh.at[pl.ds(sid * RT0, RTW)],
                    agg_out.at[cid, pl.ds(sid * RT0, RTW)])
    if with_cnt:
      pltpu.sync_copy(cnt_sh.at[pl.ds(sid * RT0, RTW)],
                      cnt_out.at[cid, pl.ds(sid * RT0, RTW)])

  return pl.kernel(
      body, out_type=out_type, mesh=mesh, scratch_types=scratch,
      compiler_params=pltpu.CompilerParams(use_tc_tiling_on_sc=False))


_seg_sum_128 = _seg_sum_sc(D, with_cnt=True, ch=CH, nch=NCH, nbuf=2)
_seg_sum_64 = _seg_sum_sc(C, with_cnt=False, ch=CH1, nch=NCH1, nbuf=4)

_TC_R = 1000


def _layer0_body(agg_ref, cnt_ref, x_ref, wl0_ref, bl0_ref, wr0_ref, wl1_ref,
                 h_ref, y1_ref):
  agg = agg_ref[0] + agg_ref[1]
  cnt = cnt_ref[0, :, 0:1] + cnt_ref[1, :, 0:1]
  mean = agg / jnp.maximum(cnt, 1.0)
  pre = (jnp.dot(mean, wl0_ref[...], preferred_element_type=jnp.float32)
         + bl0_ref[...]
         + jnp.dot(x_ref[...], wr0_ref[...], preferred_element_type=jnp.float32))
  h = jnp.maximum(pre, 0.0)
  h_ref[...] = h
  y1_ref[...] = jnp.dot(h, wl1_ref[...], preferred_element_type=jnp.float32)


def _layer1_body(agg_ref, cnt_ref, h_ref, wr1_ref, bl1_ref, out_ref):
  agg = agg_ref[0] + agg_ref[1]
  cnt = cnt_ref[0, :, 0:1] + cnt_ref[1, :, 0:1]
  pre = (agg / jnp.maximum(cnt, 1.0) + bl1_ref[...]
         + jnp.dot(h_ref[...], wr1_ref[...], preferred_element_type=jnp.float32))
  a = jnp.maximum(pre, 0.0)
  m = jnp.max(a, axis=-1, keepdims=True)
  lse = jnp.log(jnp.sum(jnp.exp(a - m), axis=-1, keepdims=True)) + m
  out_ref[...] = a - lse


_layer0 = pl.pallas_call(
    _layer0_body,
    grid=(N // _TC_R,),
    in_specs=[
        pl.BlockSpec((NC, _TC_R, D), lambda i: (0, i, 0)),
        pl.BlockSpec((NC, _TC_R, CW), lambda i: (0, i, 0)),
        pl.BlockSpec((_TC_R, D), lambda i: (i, 0)),
        pl.BlockSpec((D, D), lambda i: (0, 0)),
        pl.BlockSpec((1, D), lambda i: (0, 0)),
        pl.BlockSpec((D, D), lambda i: (0, 0)),
        pl.BlockSpec((D, C), lambda i: (0, 0)),
    ],
    out_specs=[
        pl.BlockSpec((_TC_R, D), lambda i: (i, 0)),
        pl.BlockSpec((_TC_R, C), lambda i: (i, 0)),
    ],
    out_shape=[
        jax.ShapeDtypeStruct((N, D), jnp.float32),
        jax.ShapeDtypeStruct((N, C), jnp.float32),
    ],
)

_layer1 = pl.pallas_call(
    _layer1_body,
    grid=(N // _TC_R,),
    in_specs=[
        pl.BlockSpec((NC, _TC_R, C), lambda i: (0, i, 0)),
        pl.BlockSpec((NC, _TC_R, CW), lambda i: (0, i, 0)),
        pl.BlockSpec((_TC_R, D), lambda i: (i, 0)),
        pl.BlockSpec((D, C), lambda i: (0, 0)),
        pl.BlockSpec((1, C), lambda i: (0, 0)),
    ],
    out_specs=pl.BlockSpec((_TC_R, C), lambda i: (i, 0)),
    out_shape=jax.ShapeDtypeStruct((N, C), jnp.float32),
)


@jax.jit
def kernel(x, edge_index, W_l0, b_l0, W_r0, W_l1, b_l1, W_r1):
  src = edge_index[0].reshape(NW, NCH, CH)
  dst = edge_index[1].reshape(NW, NCH, CH)
  src1 = edge_index[0].reshape(NW, NCH1, CH1)
  dst1 = edge_index[1].reshape(NW, NCH1, CH1)
  ones = jnp.ones((CH, CW), jnp.float32)
  zeros = jnp.zeros((N, CW), jnp.float32)

  agg0, cnt = _seg_sum_128(x, src, dst, ones, zeros)
  h, y1 = _layer0(agg0, cnt, x, W_l0, b_l0.reshape(1, D), W_r0, W_l1)
  (agg1,) = _seg_sum_64(y1, src1, dst1, ones, zeros)
  return _layer1(agg1, cnt, h, W_r1, b_l1.reshape(1, C))


# speedup vs baseline: 2.0088x; 1.0591x over previous
import functools

import jax
import jax.numpy as jnp
from jax import lax
from jax.experimental import pallas as pl
from jax.experimental.pallas import tpu as pltpu
from jax.experimental.pallas import tpu_sc as plsc

N = 10000
D = 128
E = 320000
C = 64

NC = 2
NS = 16
NW = NC * NS
PT = E // NW
CH = 40
NCH = PT // CH
CH1 = 125
NCH1 = PT // CH1
CW = 8
RT0 = 624
RTW = 640
ZR = 32


def _seg_sum_sc(width, with_cnt, ch, nch, nbuf=2):
  odd2 = nbuf == 2 and nch % 2 == 1
  assert odd2 or nch % nbuf == 0
  mesh = plsc.VectorSubcoreMesh(core_axis_name="c", subcore_axis_name="s")
  out_type = [jax.ShapeDtypeStruct((NC, N, width), jnp.float32)]
  scratch = [
      pltpu.VMEM((nch, ch), jnp.int32),
      pltpu.VMEM((nch, ch), jnp.int32),
  ]
  scratch += [pltpu.VMEM((ch, width), jnp.float32)] * nbuf
  scratch += [pltpu.VMEM_SHARED((N, width), jnp.float32)]
  scratch += [pltpu.SemaphoreType.DMA] * nbuf
  if with_cnt:
    out_type.append(jax.ShapeDtypeStruct((NC, N, CW), jnp.float32))
    scratch += [
        pltpu.VMEM((ch, CW), jnp.float32),
        pltpu.VMEM_SHARED((N, CW), jnp.float32),
    ]

  def body(table_hbm, src_hbm, dst_hbm, ones_hbm, zeros8_hbm, zerosw_hbm,
           *refs):
    if with_cnt:
      agg_out, cnt_out = refs[0], refs[1]
      refs = refs[2:]
    else:
      agg_out = refs[0]
      refs = refs[1:]
    src_v, dst_v = refs[0], refs[1]
    rows = list(refs[2:2 + nbuf])
    agg_sh = refs[2 + nbuf]
    sems = list(refs[3 + nbuf:3 + 2 * nbuf])
    if with_cnt:
      ones_v, cnt_sh = refs[3 + 2 * nbuf], refs[4 + 2 * nbuf]
    cid = lax.axis_index("c")
    sid = lax.axis_index("s")
    wid = cid * NS + sid

    pltpu.sync_copy(zerosw_hbm.at[pl.ds(sid * RT0, RTW)],
                    agg_sh.at[pl.ds(sid * RT0, RTW)])
    if with_cnt:
      pltpu.sync_copy(ones_hbm, ones_v)
      pltpu.sync_copy(zeros8_hbm.at[pl.ds(sid * RT0, RTW)],
                      cnt_sh.at[pl.ds(sid * RT0, RTW)])

    pltpu.sync_copy(src_hbm.at[wid], src_v)
    pltpu.sync_copy(dst_hbm.at[wid], dst_v)

    plsc.subcore_barrier()

    def start_gather(rows, sem, j):
      pltpu.async_copy(table_hbm.at[src_v.at[j]], rows, sem)

    def wait_gather(rows, sem, j):
      pltpu.make_async_copy(table_hbm.at[src_v.at[j]], rows, sem).wait()

    def scatter(rows, j):
      pltpu.sync_copy(rows, agg_sh.at[dst_v.at[j]], add=True)
      if with_cnt:
        pltpu.sync_copy(ones_v, cnt_sh.at[dst_v.at[j]], add=True)

    if odd2:
      start_gather(rows[0], sems[0], 0)

      @pl.loop(1, nch - 1, step=2)
      def _(j):
        start_gather(rows[1], sems[1], j)
        wait_gather(rows[0], sems[0], j - 1)
        scatter(rows[0], j - 1)
        start_gather(rows[0], sems[0], j + 1)
        wait_gather(rows[1], sems[1], j)
        scatter(rows[1], j)

      wait_gather(rows[0], sems[0], nch - 1)
      scatter(rows[0], nch - 1)
    else:
      for k in range(nbuf):
        start_gather(rows[k], sems[k], k)

      @pl.loop(0, nch - 2 * nbuf + 1, step=nbuf)
      def _(j):
        for k in range(nbuf):
          wait_gather(rows[k], sems[k], j + k)
          scatter(rows[k], j + k)
          start_gather(rows[k], sems[k], j + k + nbuf)

      for k in range(nbuf):
        wait_gather(rows[k], sems[k], nch - nbuf + k)
        scatter(rows[k], nch - nbuf + k)

    plsc.subcore_barrier()

    pltpu.sync_copy(agg_sh.at[pl.ds(sid * RT0, RTW)],
                    agg_out.at[cid, pl.ds(sid * RT0, RTW)])
    if with_cnt:
      pltpu.sync_copy(cnt_sh.at[pl.ds(sid * RT0, RTW)],
                      cnt_out.at[cid, pl.ds(sid * RT0, RTW)])

  return pl.kernel(
      body, out_type=out_type, mesh=mesh, scratch_types=scratch,
      compiler_params=pltpu.CompilerParams(use_tc_tiling_on_sc=False))


_seg_sum_128 = _seg_sum_sc(D, with_cnt=True, ch=CH, nch=NCH, nbuf=5)
_seg_sum_64 = _seg_sum_sc(C, with_cnt=False, ch=CH1, nch=NCH1, nbuf=4)

_TC_R = 1000


def _layer0_body(agg_ref, cnt_ref, x_ref, wl0_ref, bl0_ref, wr0_ref, wl1_ref,
                 h_ref, y1_ref):
  agg = agg_ref[0] + agg_ref[1]
  cnt = cnt_ref[0, :, 0:1] + cnt_ref[1, :, 0:1]
  mean = agg / jnp.maximum(cnt, 1.0)
  pre = (jnp.dot(mean, wl0_ref[...], preferred_element_type=jnp.float32)
         + bl0_ref[...]
         + jnp.dot(x_ref[...], wr0_ref[...], preferred_element_type=jnp.float32))
  h = jnp.maximum(pre, 0.0)
  h_ref[...] = h
  y1_ref[...] = jnp.dot(h, wl1_ref[...], preferred_element_type=jnp.float32)


def _layer1_body(agg_ref, cnt_ref, h_ref, wr1_ref, bl1_ref, out_ref):
  agg = agg_ref[0] + agg_ref[1]
  cnt = cnt_ref[0, :, 0:1] + cnt_ref[1, :, 0:1]
  pre = (agg / jnp.maximum(cnt, 1.0) + bl1_ref[...]
         + jnp.dot(h_ref[...], wr1_ref[...], preferred_element_type=jnp.float32))
  a = jnp.maximum(pre, 0.0)
  m = jnp.max(a, axis=-1, keepdims=True)
  lse = jnp.log(jnp.sum(jnp.exp(a - m), axis=-1, keepdims=True)) + m
  out_ref[...] = a - lse


_layer0 = pl.pallas_call(
    _layer0_body,
    grid=(N // _TC_R,),
    in_specs=[
        pl.BlockSpec((NC, _TC_R, D), lambda i: (0, i, 0)),
        pl.BlockSpec((NC, _TC_R, CW), lambda i: (0, i, 0)),
        pl.BlockSpec((_TC_R, D), lambda i: (i, 0)),
        pl.BlockSpec((D, D), lambda i: (0, 0)),
        pl.BlockSpec((1, D), lambda i: (0, 0)),
        pl.BlockSpec((D, D), lambda i: (0, 0)),
        pl.BlockSpec((D, C), lambda i: (0, 0)),
    ],
    out_specs=[
        pl.BlockSpec((_TC_R, D), lambda i: (i, 0)),
        pl.BlockSpec((_TC_R, C), lambda i: (i, 0)),
    ],
    out_shape=[
        jax.ShapeDtypeStruct((N, D), jnp.float32),
        jax.ShapeDtypeStruct((N, C), jnp.float32),
    ],
)

_layer1 = pl.pallas_call(
    _layer1_body,
    grid=(N // _TC_R,),
    in_specs=[
        pl.BlockSpec((NC, _TC_R, C), lambda i: (0, i, 0)),
        pl.BlockSpec((NC, _TC_R, CW), lambda i: (0, i, 0)),
        pl.BlockSpec((_TC_R, D), lambda i: (i, 0)),
        pl.BlockSpec((D, C), lambda i: (0, 0)),
        pl.BlockSpec((1, C), lambda i: (0, 0)),
    ],
    out_specs=pl.BlockSpec((_TC_R, C), lambda i: (i, 0)),
    out_shape=jax.ShapeDtypeStruct((N, C), jnp.float32),
)


@jax.jit
def kernel(x, edge_index, W_l0, b_l0, W_r0, W_l1, b_l1, W_r1):
  src = edge_index[0].reshape(NW, NCH, CH)
  dst = edge_index[1].reshape(NW, NCH, CH)
  src1 = edge_index[0].reshape(NW, NCH1, CH1)
  dst1 = edge_index[1].reshape(NW, NCH1, CH1)
  ones = jnp.ones((CH, CW), jnp.float32)
  zeros8 = jnp.zeros((N, CW), jnp.float32)
  zeros128 = jnp.zeros((N, D), jnp.float32)
  zeros64 = jnp.zeros((N, C), jnp.float32)

  agg0, cnt = _seg_sum_128(x, src, dst, ones, zeros8, zeros128)
  h, y1 = _layer0(agg0, cnt, x, W_l0, b_l0.reshape(1, D), W_r0, W_l1)
  (agg1,) = _seg_sum_64(y1, src1, dst1, ones, zeros8, zeros64)
  return _layer1(agg1, cnt, h, W_r1, b_l1.reshape(1, C))


# speedup vs baseline: 2.1508x; 1.0707x over previous
import functools

import jax
import jax.numpy as jnp
from jax import lax
from jax.experimental import pallas as pl
from jax.experimental.pallas import tpu as pltpu
from jax.experimental.pallas import tpu_sc as plsc

N = 10000
D = 128
E = 320000
C = 64

NC = 2
NS = 16
NW = NC * NS
PT = E // NW
CH = 40
NCH = PT // CH
CH1 = 80
NCH1 = PT // CH1
CW = 8
RT0 = 624
RTW = 640
ZR = 32


def _seg_sum_sc(width, with_cnt, ch, nch, nbuf=2):
  odd2 = nbuf == 2 and nch % 2 == 1
  assert odd2 or nch % nbuf == 0
  assert ch % 8 == 0
  mesh = plsc.VectorSubcoreMesh(core_axis_name="c", subcore_axis_name="s")
  out_type = [jax.ShapeDtypeStruct((NC, N, width), jnp.float32)]
  scratch = [
      pltpu.VMEM((PT,), jnp.int32),
      pltpu.VMEM((PT,), jnp.int32),
  ]
  scratch += [pltpu.VMEM((ch, width), jnp.float32)] * nbuf
  scratch += [pltpu.VMEM_SHARED((N, width), jnp.float32)]
  scratch += [pltpu.SemaphoreType.DMA] * nbuf
  if with_cnt:
    out_type.append(jax.ShapeDtypeStruct((NC, N, CW), jnp.float32))
    scratch += [
        pltpu.VMEM((ch, CW), jnp.float32),
        pltpu.VMEM_SHARED((N, CW), jnp.float32),
    ]

  def body(table_hbm, ei_hbm, ones_hbm, zeros8_hbm, zerosw_hbm, *refs):
    if with_cnt:
      agg_out, cnt_out = refs[0], refs[1]
      refs = refs[2:]
    else:
      agg_out = refs[0]
      refs = refs[1:]
    src_v, dst_v = refs[0], refs[1]
    rows = list(refs[2:2 + nbuf])
    agg_sh = refs[2 + nbuf]
    sems = list(refs[3 + nbuf:3 + 2 * nbuf])
    if with_cnt:
      ones_v, cnt_sh = refs[3 + 2 * nbuf], refs[4 + 2 * nbuf]
    cid = lax.axis_index("c")
    sid = lax.axis_index("s")
    wid = cid * NS + sid

    pltpu.sync_copy(zerosw_hbm.at[pl.ds(sid * RT0, RTW)],
                    agg_sh.at[pl.ds(sid * RT0, RTW)])
    if with_cnt:
      pltpu.sync_copy(ones_hbm, ones_v)
      pltpu.sync_copy(zeros8_hbm.at[pl.ds(sid * RT0, RTW)],
                      cnt_sh.at[pl.ds(sid * RT0, RTW)])

    pltpu.sync_copy(ei_hbm.at[0, pl.ds(wid * PT, PT)], src_v)
    pltpu.sync_copy(ei_hbm.at[1, pl.ds(wid * PT, PT)], dst_v)

    plsc.subcore_barrier()

    def start_gather(rows, sem, j):
      pltpu.async_copy(table_hbm.at[src_v.at[pl.ds(j * ch, ch)]], rows, sem)

    def wait_gather(rows, sem, j):
      pltpu.make_async_copy(
          table_hbm.at[src_v.at[pl.ds(j * ch, ch)]], rows, sem).wait()

    def scatter(rows, j):
      pltpu.sync_copy(rows, agg_sh.at[dst_v.at[pl.ds(j * ch, ch)]], add=True)
      if with_cnt:
        pltpu.sync_copy(ones_v, cnt_sh.at[dst_v.at[pl.ds(j * ch, ch)]],
                        add=True)

    if odd2:
      start_gather(rows[0], sems[0], 0)

      @pl.loop(1, nch - 1, step=2)
      def _(j):
        start_gather(rows[1], sems[1], j)
        wait_gather(rows[0], sems[0], j - 1)
        scatter(rows[0], j - 1)
        start_gather(rows[0], sems[0], j + 1)
        wait_gather(rows[1], sems[1], j)
        scatter(rows[1], j)

      wait_gather(rows[0], sems[0], nch - 1)
      scatter(rows[0], nch - 1)
    else:
      for k in range(nbuf):
        start_gather(rows[k], sems[k], k)

      @pl.loop(0, nch - 2 * nbuf + 1, step=nbuf)
      def _(j):
        for k in range(nbuf):
          wait_gather(rows[k], sems[k], j + k)
          scatter(rows[k], j + k)
          start_gather(rows[k], sems[k], j + k + nbuf)

      for k in range(nbuf):
        wait_gather(rows[k], sems[k], nch - nbuf + k)
        scatter(rows[k], nch - nbuf + k)

    plsc.subcore_barrier()

    pltpu.sync_copy(agg_sh.at[pl.ds(sid * RT0, RTW)],
                    agg_out.at[cid, pl.ds(sid * RT0, RTW)])
    if with_cnt:
      pltpu.sync_copy(cnt_sh.at[pl.ds(sid * RT0, RTW)],
                      cnt_out.at[cid, pl.ds(sid * RT0, RTW)])

  return pl.kernel(
      body, out_type=out_type, mesh=mesh, scratch_types=scratch,
      compiler_params=pltpu.CompilerParams(use_tc_tiling_on_sc=False))


_seg_sum_128 = _seg_sum_sc(D, with_cnt=True, ch=CH, nch=NCH, nbuf=5)
_seg_sum_64 = _seg_sum_sc(C, with_cnt=False, ch=CH1, nch=NCH1, nbuf=5)

_TC_R = 2000


def _layer0_body(agg_ref, cnt_ref, x_ref, wl0_ref, bl0_ref, wr0_ref, wl1_ref,
                 h_ref, y1_ref):
  agg = agg_ref[0] + agg_ref[1]
  cnt = cnt_ref[0, :, 0:1] + cnt_ref[1, :, 0:1]
  mean = agg / jnp.maximum(cnt, 1.0)
  pre = (jnp.dot(mean, wl0_ref[...], preferred_element_type=jnp.float32)
         + bl0_ref[...]
         + jnp.dot(x_ref[...], wr0_ref[...], preferred_element_type=jnp.float32))
  h = jnp.maximum(pre, 0.0)
  h_ref[...] = h
  y1_ref[...] = jnp.dot(h, wl1_ref[...], preferred_element_type=jnp.float32)


def _layer1_body(agg_ref, cnt_ref, h_ref, wr1_ref, bl1_ref, out_ref):
  agg = agg_ref[0] + agg_ref[1]
  cnt = cnt_ref[0, :, 0:1] + cnt_ref[1, :, 0:1]
  pre = (agg / jnp.maximum(cnt, 1.0) + bl1_ref[...]
         + jnp.dot(h_ref[...], wr1_ref[...], preferred_element_type=jnp.float32))
  a = jnp.maximum(pre, 0.0)
  m = jnp.max(a, axis=-1, keepdims=True)
  lse = jnp.log(jnp.sum(jnp.exp(a - m), axis=-1, keepdims=True)) + m
  out_ref[...] = a - lse


_layer0 = pl.pallas_call(
    _layer0_body,
    grid=(N // _TC_R,),
    in_specs=[
        pl.BlockSpec((NC, _TC_R, D), lambda i: (0, i, 0)),
        pl.BlockSpec((NC, _TC_R, CW), lambda i: (0, i, 0)),
        pl.BlockSpec((_TC_R, D), lambda i: (i, 0)),
        pl.BlockSpec((D, D), lambda i: (0, 0)),
        pl.BlockSpec((1, D), lambda i: (0, 0)),
        pl.BlockSpec((D, D), lambda i: (0, 0)),
        pl.BlockSpec((D, C), lambda i: (0, 0)),
    ],
    out_specs=[
        pl.BlockSpec((_TC_R, D), lambda i: (i, 0)),
        pl.BlockSpec((_TC_R, C), lambda i: (i, 0)),
    ],
    out_shape=[
        jax.ShapeDtypeStruct((N, D), jnp.float32),
        jax.ShapeDtypeStruct((N, C), jnp.float32),
    ],
)

_layer1 = pl.pallas_call(
    _layer1_body,
    grid=(N // _TC_R,),
    in_specs=[
        pl.BlockSpec((NC, _TC_R, C), lambda i: (0, i, 0)),
        pl.BlockSpec((NC, _TC_R, CW), lambda i: (0, i, 0)),
        pl.BlockSpec((_TC_R, D), lambda i: (i, 0)),
        pl.BlockSpec((D, C), lambda i: (0, 0)),
        pl.BlockSpec((1, C), lambda i: (0, 0)),
    ],
    out_specs=pl.BlockSpec((_TC_R, C), lambda i: (i, 0)),
    out_shape=jax.ShapeDtypeStruct((N, C), jnp.float32),
)


@jax.jit
def kernel(x, edge_index, W_l0, b_l0, W_r0, W_l1, b_l1, W_r1):
  ones = jnp.ones((CH, CW), jnp.float32)
  zeros8 = jnp.zeros((N, CW), jnp.float32)
  zeros128 = jnp.zeros((N, D), jnp.float32)
  zeros64 = jnp.zeros((N, C), jnp.float32)

  agg0, cnt = _seg_sum_128(x, edge_index, ones, zeros8, zeros128)
  h, y1 = _layer0(agg0, cnt, x, W_l0, b_l0.reshape(1, D), W_r0, W_l1)
  (agg1,) = _seg_sum_64(y1, edge_index, ones, zeros8, zeros64)
  return _layer1(agg1, cnt, h, W_r1, b_l1.reshape(1, C))
